# baseline (speedup 1.0000x reference)
import functools

import jax
import jax.numpy as jnp
from jax import lax
from jax.experimental import pallas as pl
from jax.experimental.pallas import tpu as pltpu
from jax.experimental.pallas import tpu_sc as plsc

N_NODES = 10000
N_EDGES = 320000
D = 128

NC = 2
NS = 16
NW = NC * NS

NP = 10240
EP = 327680
EPT = EP // NW
CHUNK = 128
NCHUNK = EPT // CHUNK
GRP = 4
NGRP = NCHUNK // GRP
NIB = 3
RPT = NP // NS


def _seg_body(x_hbm, src_hbm, dst_hbm, z_hbm, out_hbm,
              src_g, dst_g, rows_a, rows_b, gs0, gs1, ss0, ss1, acc_sh):
    c = lax.axis_index("c")
    s = lax.axis_index("s")
    wid = c * NS + s
    rows = (rows_a, rows_b)
    gsem = (gs0, gs1)
    ssem = (ss0, ss1)

    pltpu.sync_copy(z_hbm, acc_sh.at[pl.ds(s * RPT, RPT)])

    def stage(g):
        e = g % NIB
        pltpu.sync_copy(
            src_hbm.at[pl.ds(wid * EPT + g * GRP * CHUNK, GRP * CHUNK)],
            src_g.at[pl.ds(e * GRP * CHUNK, GRP * CHUNK)])
        pltpu.sync_copy(dst_hbm.at[pl.ds(wid * NCHUNK + g * GRP, GRP)],
                        dst_g.at[pl.ds(e * GRP, GRP)])

    stage(0)
    plsc.subcore_barrier()
    gd = [None, None]
    sd = [None, None]
    for g in range(NGRP):
        if g + 1 < NGRP:
            stage(g + 1)
        for i in range(GRP):
            b = g * GRP + i
            p = b & 1
            if sd[p] is not None:
                sd[p].wait()
            gd[p] = pltpu.async_copy(
                x_hbm.at[src_g.at[pl.ds((g % NIB) * GRP * CHUNK + i * CHUNK, CHUNK)]],
                rows[p], gsem[p])
            if b > 0:
                bp = b - 1
                q = bp & 1
                gd[q].wait()
                sd[q] = pltpu.async_copy(
                    rows[q], acc_sh.at[dst_g.at[((bp // GRP) % NIB) * GRP + bp % GRP]],
                    ssem[q], add=True)
    bp = NCHUNK - 1
    q = bp & 1
    gd[q].wait()
    sd[q] = pltpu.async_copy(
        rows[q], acc_sh.at[dst_g.at[((bp // GRP) % NIB) * GRP + bp % GRP]], ssem[q], add=True)
    sd[0].wait()
    sd[1].wait()
    plsc.subcore_barrier()

    pltpu.sync_copy(acc_sh.at[pl.ds(s * RPT, RPT)], out_hbm.at[c, pl.ds(s * RPT, RPT)])


def _cnt_body(dst_hbm, z_hbm, ones_hbm, cnt_hbm, dst_g, ones_v, cs, cnt_sh):
    c = lax.axis_index("c")
    s = lax.axis_index("s")
    wid = c * NS + s

    pltpu.sync_copy(z_hbm, cnt_sh.at[pl.ds(s * RPT, RPT)])
    pltpu.sync_copy(ones_hbm, ones_v)
    plsc.subcore_barrier()

    pltpu.sync_copy(dst_hbm.at[pl.ds(wid * NCHUNK, NCHUNK)], dst_g)
    descs = []
    for b in range(NCHUNK):
        if b >= 8:
            descs[b - 8].wait()
        descs.append(
            pltpu.async_copy(ones_v, cnt_sh.at[dst_g.at[b]], cs, add=True))
    for d in descs[-8:]:
        d.wait()
    plsc.subcore_barrier()

    pltpu.sync_copy(cnt_sh.at[pl.ds(s * RPT, RPT)], cnt_hbm.at[c, pl.ds(s * RPT, RPT)])


def _sc_mesh():
    return plsc.VectorSubcoreMesh(
        core_axis_name="c", subcore_axis_name="s", num_cores=NC, num_subcores=NS
    )


@functools.lru_cache(maxsize=None)
def _make_seg_sum():
    return pl.kernel(
        _seg_body,
        out_type=[jax.ShapeDtypeStruct((NC, NP, D), jnp.float32)],
        mesh=_sc_mesh(),
        scratch_types=[
            pltpu.VMEM((NIB * GRP * CHUNK,), jnp.int32),
            pltpu.VMEM((NIB * GRP, CHUNK), jnp.int32),
            pltpu.VMEM((CHUNK, D), jnp.float32),
            pltpu.VMEM((CHUNK, D), jnp.float32),
            pltpu.SemaphoreType.DMA,
            pltpu.SemaphoreType.DMA,
            pltpu.SemaphoreType.DMA,
            pltpu.SemaphoreType.DMA,
            pltpu.VMEM_SHARED((NP, D), jnp.float32),
        ],
    )


@functools.lru_cache(maxsize=None)
def _make_counts():
    return pl.kernel(
        _cnt_body,
        out_type=[jax.ShapeDtypeStruct((NC, NP, D), jnp.float32)],
        mesh=_sc_mesh(),
        scratch_types=[
            pltpu.VMEM((NCHUNK, CHUNK), jnp.int32),
            pltpu.VMEM((CHUNK, D), jnp.float32),
            pltpu.SemaphoreType.DMA,
            pltpu.VMEM_SHARED((NP, D), jnp.float32),
        ],
    )


def _tc1_body(parts_ref, cnts_ref, x_ref, wl_ref, bl_ref, wr_ref,
              out_ref, inv_ref):
    summed = parts_ref[0] + parts_ref[1]
    cnt = cnts_ref[0, :, 0:1] + cnts_ref[1, :, 0:1]
    inv = 1.0 / jnp.maximum(cnt, 1.0)
    inv_ref[...] = inv
    mean = summed * inv
    h = (jnp.dot(mean, wl_ref[...], preferred_element_type=jnp.float32)
         + bl_ref[...]
         + jnp.dot(x_ref[...], wr_ref[...], preferred_element_type=jnp.float32))
    out_ref[...] = jnp.maximum(h, 0.0)


def _tc2_body(parts_ref, inv_ref, x_ref, wl_ref, bl_ref, wr_ref, out_ref):
    summed = parts_ref[0] + parts_ref[1]
    mean = summed * inv_ref[...]
    out_ref[...] = (jnp.dot(mean, wl_ref[...], preferred_element_type=jnp.float32)
                    + bl_ref[...]
                    + jnp.dot(x_ref[...], wr_ref[...],
                              preferred_element_type=jnp.float32))


BN = 1024


def _sage_tc1(parts, cnts, x, wl, bl, wr):
    grid = NP // BN
    return pl.pallas_call(
        _tc1_body,
        grid=(grid,),
        in_specs=[
            pl.BlockSpec((NC, BN, D), lambda j: (0, j, 0)),
            pl.BlockSpec((NC, BN, D), lambda j: (0, j, 0)),
            pl.BlockSpec((BN, D), lambda j: (j, 0)),
            pl.BlockSpec((D, D), lambda j: (0, 0)),
            pl.BlockSpec((1, D), lambda j: (0, 0)),
            pl.BlockSpec((D, D), lambda j: (0, 0)),
        ],
        out_specs=[
            pl.BlockSpec((BN, D), lambda j: (j, 0)),
            pl.BlockSpec((BN, 1), lambda j: (j, 0)),
        ],
        out_shape=[
            jax.ShapeDtypeStruct((N_NODES, D), jnp.float32),
            jax.ShapeDtypeStruct((NP, 1), jnp.float32),
        ],
    )(parts, cnts, x, wl, bl.reshape(1, D), wr)


def _sage_tc2(parts, inv, x, wl, bl, wr):
    grid = NP // BN
    return pl.pallas_call(
        _tc2_body,
        grid=(grid,),
        in_specs=[
            pl.BlockSpec((NC, BN, D), lambda j: (0, j, 0)),
            pl.BlockSpec((BN, 1), lambda j: (j, 0)),
            pl.BlockSpec((BN, D), lambda j: (j, 0)),
            pl.BlockSpec((D, D), lambda j: (0, 0)),
            pl.BlockSpec((1, D), lambda j: (0, 0)),
            pl.BlockSpec((D, D), lambda j: (0, 0)),
        ],
        out_specs=pl.BlockSpec((BN, D), lambda j: (j, 0)),
        out_shape=jax.ShapeDtypeStruct((N_NODES, D), jnp.float32),
    )(parts, inv, x, wl, bl.reshape(1, D), wr)


def kernel(x, edge_index, Wl1, bl1, Wr1, Wl2, bl2, Wr2):
    src = edge_index[0]
    dst = edge_index[1]
    pad = EP - N_EDGES
    pad_ids = jnp.arange(pad, dtype=jnp.int32)
    src_p = jnp.concatenate([src, pad_ids % N_NODES])
    dst_p = jnp.concatenate([dst, N_NODES + pad_ids % (NP - N_NODES)])
    dst2d = dst_p.reshape(EP // CHUNK, CHUNK)

    zeros_rows = jnp.zeros((RPT, D), jnp.float32)
    ones_rows = jnp.ones((CHUNK, D), jnp.float32)

    cnts = _make_counts()(dst2d, zeros_rows, ones_rows)[0]
    parts1 = _make_seg_sum()(x, src_p, dst2d, zeros_rows)[0]
    h, inv = _sage_tc1(parts1, cnts, x, Wl1, bl1, Wr1)
    parts2 = _make_seg_sum()(h, src_p, dst2d, zeros_rows)[0]
    out = _sage_tc2(parts2, inv, h, Wl2, bl2, Wr2)
    return out


# speedup vs baseline: 1.0016x; 1.0016x over previous
import functools

import jax
import jax.numpy as jnp
from jax import lax
from jax.experimental import pallas as pl
from jax.experimental.pallas import tpu as pltpu
from jax.experimental.pallas import tpu_sc as plsc

N_NODES = 10000
N_EDGES = 320000
D = 128

NC = 2
NS = 16
NW = NC * NS

NP = 10240
EP = 327680
EPT = EP // NW
CHUNK = 128
NCHUNK = EPT // CHUNK
GRP = 4
NGRP = NCHUNK // GRP
NIB = 3
RPT = NP // NS


def _seg_body(x_hbm, src_hbm, dst_hbm, z_hbm, out_hbm,
              src_g, dst_g, rows_a, rows_b, gs0, gs1, ss0, ss1, acc_sh):
    c = lax.axis_index("c")
    s = lax.axis_index("s")
    wid = c * NS + s
    rows = (rows_a, rows_b)
    gsem = (gs0, gs1)
    ssem = (ss0, ss1)

    pltpu.sync_copy(z_hbm, acc_sh.at[pl.ds(s * RPT, RPT)])
    plsc.subcore_barrier()

    def stage(g):
        e = g % NIB
        pltpu.sync_copy(
            src_hbm.at[pl.ds(wid * EPT + g * GRP * CHUNK, GRP * CHUNK)],
            src_g.at[pl.ds(e * GRP * CHUNK, GRP * CHUNK)])
        pltpu.sync_copy(dst_hbm.at[pl.ds(wid * NCHUNK + g * GRP, GRP)],
                        dst_g.at[pl.ds(e * GRP, GRP)])

    stage(0)
    gd = [None, None]
    sd = [None, None]
    for g in range(NGRP):
        if g + 1 < NGRP:
            stage(g + 1)
        for i in range(GRP):
            b = g * GRP + i
            p = b & 1
            if sd[p] is not None:
                sd[p].wait()
            gd[p] = pltpu.async_copy(
                x_hbm.at[src_g.at[pl.ds((g % NIB) * GRP * CHUNK + i * CHUNK, CHUNK)]],
                rows[p], gsem[p])
            if b > 0:
                bp = b - 1
                q = bp & 1
                gd[q].wait()
                sd[q] = pltpu.async_copy(
                    rows[q], acc_sh.at[dst_g.at[((bp // GRP) % NIB) * GRP + bp % GRP]],
                    ssem[q], add=True)
    bp = NCHUNK - 1
    q = bp & 1
    gd[q].wait()
    sd[q] = pltpu.async_copy(
        rows[q], acc_sh.at[dst_g.at[((bp // GRP) % NIB) * GRP + bp % GRP]], ssem[q], add=True)
    sd[0].wait()
    sd[1].wait()
    plsc.subcore_barrier()

    pltpu.sync_copy(acc_sh.at[pl.ds(s * RPT, RPT)], out_hbm.at[c, pl.ds(s * RPT, RPT)])


def _cnt_body(dst_hbm, z_hbm, ones_hbm, cnt_hbm, dst_g, ones_v, cs, cnt_sh):
    c = lax.axis_index("c")
    s = lax.axis_index("s")
    wid = c * NS + s

    pltpu.sync_copy(z_hbm, cnt_sh.at[pl.ds(s * RPT, RPT)])
    pltpu.sync_copy(ones_hbm, ones_v)
    plsc.subcore_barrier()

    pltpu.sync_copy(dst_hbm.at[pl.ds(wid * NCHUNK, NCHUNK)], dst_g)
    descs = []
    for b in range(NCHUNK):
        if b >= 8:
            descs[b - 8].wait()
        descs.append(
            pltpu.async_copy(ones_v, cnt_sh.at[dst_g.at[b]], cs, add=True))
    for d in descs[-8:]:
        d.wait()
    plsc.subcore_barrier()

    pltpu.sync_copy(cnt_sh.at[pl.ds(s * RPT, RPT)], cnt_hbm.at[c, pl.ds(s * RPT, RPT)])


def _sc_mesh():
    return plsc.VectorSubcoreMesh(
        core_axis_name="c", subcore_axis_name="s", num_cores=NC, num_subcores=NS
    )


@functools.lru_cache(maxsize=None)
def _make_seg_sum():
    return pl.kernel(
        _seg_body,
        out_type=[jax.ShapeDtypeStruct((NC, NP, D), jnp.float32)],
        mesh=_sc_mesh(),
        scratch_types=[
            pltpu.VMEM((NIB * GRP * CHUNK,), jnp.int32),
            pltpu.VMEM((NIB * GRP, CHUNK), jnp.int32),
            pltpu.VMEM((CHUNK, D), jnp.float32),
            pltpu.VMEM((CHUNK, D), jnp.float32),
            pltpu.SemaphoreType.DMA,
            pltpu.SemaphoreType.DMA,
            pltpu.SemaphoreType.DMA,
            pltpu.SemaphoreType.DMA,
            pltpu.VMEM_SHARED((NP, D), jnp.float32),
        ],
    )


@functools.lru_cache(maxsize=None)
def _make_counts():
    return pl.kernel(
        _cnt_body,
        out_type=[jax.ShapeDtypeStruct((NC, NP, D), jnp.float32)],
        mesh=_sc_mesh(),
        scratch_types=[
            pltpu.VMEM((NCHUNK, CHUNK), jnp.int32),
            pltpu.VMEM((CHUNK, D), jnp.float32),
            pltpu.SemaphoreType.DMA,
            pltpu.VMEM_SHARED((NP, D), jnp.float32),
        ],
    )


def _tc_body(relu, parts_ref, cnts_ref, x_ref, wl_ref, bl_ref, wr_ref, out_ref):
    summed = parts_ref[0] + parts_ref[1]
    cnt = cnts_ref[0] + cnts_ref[1]
    mean = summed * (1.0 / jnp.maximum(cnt, 1.0))
    h = (jnp.dot(mean, wl_ref[...], preferred_element_type=jnp.float32)
         + bl_ref[...]
         + jnp.dot(x_ref[...], wr_ref[...], preferred_element_type=jnp.float32))
    out_ref[...] = jnp.maximum(h, 0.0) if relu else h


BN = 1024


def _sage_tc(parts, cnts, x, wl, bl, wr, relu):
    grid = NP // BN
    return pl.pallas_call(
        functools.partial(_tc_body, relu),
        grid=(grid,),
        in_specs=[
            pl.BlockSpec((NC, BN, D), lambda j: (0, j, 0)),
            pl.BlockSpec((NC, BN, 1), lambda j: (0, j, 0)),
            pl.BlockSpec((BN, D), lambda j: (j, 0)),
            pl.BlockSpec((D, D), lambda j: (0, 0)),
            pl.BlockSpec((1, D), lambda j: (0, 0)),
            pl.BlockSpec((D, D), lambda j: (0, 0)),
        ],
        out_specs=pl.BlockSpec((BN, D), lambda j: (j, 0)),
        out_shape=jax.ShapeDtypeStruct((N_NODES, D), jnp.float32),
    )(parts, cnts, x, wl, bl.reshape(1, D), wr)


def kernel(x, edge_index, Wl1, bl1, Wr1, Wl2, bl2, Wr2):
    src = edge_index[0]
    dst = edge_index[1]
    pad = EP - N_EDGES
    pad_ids = jnp.arange(pad, dtype=jnp.int32)
    src_p = jnp.concatenate([src, pad_ids % N_NODES])
    dst_p = jnp.concatenate([dst, N_NODES + pad_ids % (NP - N_NODES)])
    dst2d = dst_p.reshape(EP // CHUNK, CHUNK)

    zeros_rows = jnp.zeros((RPT, D), jnp.float32)
    ones_rows = jnp.ones((CHUNK, D), jnp.float32)

    cnts = _make_counts()(dst2d, zeros_rows, ones_rows)[0][:, :, 0:1]
    parts1 = _make_seg_sum()(x, src_p, dst2d, zeros_rows)[0]
    h = _sage_tc(parts1, cnts, x, Wl1, bl1, Wr1, relu=True)
    parts2 = _make_seg_sum()(h, src_p, dst2d, zeros_rows)[0]
    out = _sage_tc(parts2, cnts, h, Wl2, bl2, Wr2, relu=False)
    return out


# speedup vs baseline: 1.0239x; 1.0224x over previous
import functools

import jax
import jax.numpy as jnp
from jax import lax
from jax.experimental import pallas as pl
from jax.experimental.pallas import tpu as pltpu
from jax.experimental.pallas import tpu_sc as plsc

N_NODES = 10000
N_EDGES = 320000
D = 128

NC = 2
NS = 16
NW = NC * NS

NP = 10240
EP = 327680
EPT = EP // NW
CHUNK = 128
NCHUNK = EPT // CHUNK
GRP = 8
NGRP = NCHUNK // GRP
NIB = 3
RPT = NP // NS


def _seg_body(x_hbm, src_hbm, dst_hbm, z_hbm, out_hbm,
              src_g, dst_g, rows_a, rows_b, gs0, gs1, ss0, ss1, acc_sh):
    c = lax.axis_index("c")
    s = lax.axis_index("s")
    wid = c * NS + s
    rows = (rows_a, rows_b)
    gsem = (gs0, gs1)
    ssem = (ss0, ss1)

    pltpu.sync_copy(z_hbm, acc_sh.at[pl.ds(s * RPT, RPT)])
    plsc.subcore_barrier()

    def stage(g):
        e = g % NIB
        pltpu.sync_copy(
            src_hbm.at[pl.ds(wid * EPT + g * GRP * CHUNK, GRP * CHUNK)],
            src_g.at[pl.ds(e * GRP * CHUNK, GRP * CHUNK)])
        pltpu.sync_copy(dst_hbm.at[pl.ds(wid * NCHUNK + g * GRP, GRP)],
                        dst_g.at[pl.ds(e * GRP, GRP)])

    stage(0)
    gd = [None, None]
    sd = [None, None]
    for g in range(NGRP):
        if g + 1 < NGRP:
            stage(g + 1)
        for i in range(GRP):
            b = g * GRP + i
            p = b & 1
            if sd[p] is not None:
                sd[p].wait()
            gd[p] = pltpu.async_copy(
                x_hbm.at[src_g.at[pl.ds((g % NIB) * GRP * CHUNK + i * CHUNK, CHUNK)]],
                rows[p], gsem[p])
            if b > 0:
                bp = b - 1
                q = bp & 1
                gd[q].wait()
                sd[q] = pltpu.async_copy(
                    rows[q], acc_sh.at[dst_g.at[((bp // GRP) % NIB) * GRP + bp % GRP]],
                    ssem[q], add=True)
    bp = NCHUNK - 1
    q = bp & 1
    gd[q].wait()
    sd[q] = pltpu.async_copy(
        rows[q], acc_sh.at[dst_g.at[((bp // GRP) % NIB) * GRP + bp % GRP]], ssem[q], add=True)
    sd[0].wait()
    sd[1].wait()
    plsc.subcore_barrier()

    pltpu.sync_copy(acc_sh.at[pl.ds(s * RPT, RPT)], out_hbm.at[c, pl.ds(s * RPT, RPT)])


def _cnt_body(dst_hbm, z_hbm, ones_hbm, cnt_hbm, dst_g, ones_v, cs, cnt_sh):
    c = lax.axis_index("c")
    s = lax.axis_index("s")
    wid = c * NS + s

    pltpu.sync_copy(z_hbm, cnt_sh.at[pl.ds(s * RPT, RPT)])
    pltpu.sync_copy(ones_hbm, ones_v)
    plsc.subcore_barrier()

    pltpu.sync_copy(dst_hbm.at[pl.ds(wid * NCHUNK, NCHUNK)], dst_g)
    descs = []
    for b in range(NCHUNK):
        if b >= 8:
            descs[b - 8].wait()
        descs.append(
            pltpu.async_copy(ones_v, cnt_sh.at[dst_g.at[b]], cs, add=True))
    for d in descs[-8:]:
        d.wait()
    plsc.subcore_barrier()

    pltpu.sync_copy(cnt_sh.at[pl.ds(s * RPT, RPT)], cnt_hbm.at[c, pl.ds(s * RPT, RPT)])


def _sc_mesh():
    return plsc.VectorSubcoreMesh(
        core_axis_name="c", subcore_axis_name="s", num_cores=NC, num_subcores=NS
    )


@functools.lru_cache(maxsize=None)
def _make_seg_sum():
    return pl.kernel(
        _seg_body,
        out_type=[jax.ShapeDtypeStruct((NC, NP, D), jnp.float32)],
        mesh=_sc_mesh(),
        scratch_types=[
            pltpu.VMEM((NIB * GRP * CHUNK,), jnp.int32),
            pltpu.VMEM((NIB * GRP, CHUNK), jnp.int32),
            pltpu.VMEM((CHUNK, D), jnp.float32),
            pltpu.VMEM((CHUNK, D), jnp.float32),
            pltpu.SemaphoreType.DMA,
            pltpu.SemaphoreType.DMA,
            pltpu.SemaphoreType.DMA,
            pltpu.SemaphoreType.DMA,
            pltpu.VMEM_SHARED((NP, D), jnp.float32),
        ],
    )


@functools.lru_cache(maxsize=None)
def _make_counts():
    return pl.kernel(
        _cnt_body,
        out_type=[jax.ShapeDtypeStruct((NC, NP, D), jnp.float32)],
        mesh=_sc_mesh(),
        scratch_types=[
            pltpu.VMEM((NCHUNK, CHUNK), jnp.int32),
            pltpu.VMEM((CHUNK, D), jnp.float32),
            pltpu.SemaphoreType.DMA,
            pltpu.VMEM_SHARED((NP, D), jnp.float32),
        ],
    )


def _tc_body(relu, parts_ref, cnts_ref, x_ref, wl_ref, bl_ref, wr_ref, out_ref):
    summed = parts_ref[0] + parts_ref[1]
    cnt = cnts_ref[0] + cnts_ref[1]
    mean = summed * (1.0 / jnp.maximum(cnt, 1.0))
    h = (jnp.dot(mean, wl_ref[...], preferred_element_type=jnp.float32)
         + bl_ref[...]
         + jnp.dot(x_ref[...], wr_ref[...], preferred_element_type=jnp.float32))
    out_ref[...] = jnp.maximum(h, 0.0) if relu else h


BN = 1024


def _sage_tc(parts, cnts, x, wl, bl, wr, relu):
    grid = NP // BN
    return pl.pallas_call(
        functools.partial(_tc_body, relu),
        grid=(grid,),
        in_specs=[
            pl.BlockSpec((NC, BN, D), lambda j: (0, j, 0)),
            pl.BlockSpec((NC, BN, 1), lambda j: (0, j, 0)),
            pl.BlockSpec((BN, D), lambda j: (j, 0)),
            pl.BlockSpec((D, D), lambda j: (0, 0)),
            pl.BlockSpec((1, D), lambda j: (0, 0)),
            pl.BlockSpec((D, D), lambda j: (0, 0)),
        ],
        out_specs=pl.BlockSpec((BN, D), lambda j: (j, 0)),
        out_shape=jax.ShapeDtypeStruct((N_NODES, D), jnp.float32),
    )(parts, cnts, x, wl, bl.reshape(1, D), wr)


def kernel(x, edge_index, Wl1, bl1, Wr1, Wl2, bl2, Wr2):
    src = edge_index[0]
    dst = edge_index[1]
    pad = EP - N_EDGES
    pad_ids = jnp.arange(pad, dtype=jnp.int32)
    src_p = jnp.concatenate([src, pad_ids % N_NODES])
    dst_p = jnp.concatenate([dst, N_NODES + pad_ids % (NP - N_NODES)])
    dst2d = dst_p.reshape(EP // CHUNK, CHUNK)

    zeros_rows = jnp.zeros((RPT, D), jnp.float32)
    ones_rows = jnp.ones((CHUNK, D), jnp.float32)

    cnts = _make_counts()(dst2d, zeros_rows, ones_rows)[0][:, :, 0:1]
    parts1 = _make_seg_sum()(x, src_p, dst2d, zeros_rows)[0]
    h = _sage_tc(parts1, cnts, x, Wl1, bl1, Wr1, relu=True)
    parts2 = _make_seg_sum()(h, src_p, dst2d, zeros_rows)[0]
    out = _sage_tc(parts2, cnts, h, Wl2, bl2, Wr2, relu=False)
    return out


# speedup vs baseline: 1.0398x; 1.0154x over previous
import functools

import jax
import jax.numpy as jnp
from jax import lax
from jax.experimental import pallas as pl
from jax.experimental.pallas import tpu as pltpu
from jax.experimental.pallas import tpu_sc as plsc

N_NODES = 10000
N_EDGES = 320000
D = 128

NC = 2
NS = 16
NW = NC * NS

NP = 10240
EP = 327680
EPT = EP // NW
CHUNK = 128
NCHUNK = EPT // CHUNK
GRP = 16
NGRP = NCHUNK // GRP
NIB = 3
RPT = NP // NS


def _seg_body(x_hbm, src_hbm, dst_hbm, z_hbm, out_hbm,
              src_g, dst_g, rows_a, rows_b, gs0, gs1, ss0, ss1, acc_sh):
    c = lax.axis_index("c")
    s = lax.axis_index("s")
    wid = c * NS + s
    rows = (rows_a, rows_b)
    gsem = (gs0, gs1)
    ssem = (ss0, ss1)

    pltpu.sync_copy(z_hbm, acc_sh.at[pl.ds(s * RPT, RPT)])
    plsc.subcore_barrier()

    def stage(g):
        e = g % NIB
        pltpu.sync_copy(
            src_hbm.at[pl.ds(wid * EPT + g * GRP * CHUNK, GRP * CHUNK)],
            src_g.at[pl.ds(e * GRP * CHUNK, GRP * CHUNK)])
        pltpu.sync_copy(dst_hbm.at[pl.ds(wid * NCHUNK + g * GRP, GRP)],
                        dst_g.at[pl.ds(e * GRP, GRP)])

    stage(0)
    gd = [None, None]
    sd = [None, None]
    for g in range(NGRP):
        if g + 1 < NGRP:
            stage(g + 1)
        for i in range(GRP):
            b = g * GRP + i
            p = b & 1
            if sd[p] is not None:
                sd[p].wait()
            gd[p] = pltpu.async_copy(
                x_hbm.at[src_g.at[pl.ds((g % NIB) * GRP * CHUNK + i * CHUNK, CHUNK)]],
                rows[p], gsem[p])
            if b > 0:
                bp = b - 1
                q = bp & 1
                gd[q].wait()
                sd[q] = pltpu.async_copy(
                    rows[q], acc_sh.at[dst_g.at[((bp // GRP) % NIB) * GRP + bp % GRP]],
                    ssem[q], add=True)
    bp = NCHUNK - 1
    q = bp & 1
    gd[q].wait()
    sd[q] = pltpu.async_copy(
        rows[q], acc_sh.at[dst_g.at[((bp // GRP) % NIB) * GRP + bp % GRP]], ssem[q], add=True)
    sd[0].wait()
    sd[1].wait()
    plsc.subcore_barrier()

    pltpu.sync_copy(acc_sh.at[pl.ds(s * RPT, RPT)], out_hbm.at[c, pl.ds(s * RPT, RPT)])


def _cnt_body(dst_hbm, z_hbm, ones_hbm, cnt_hbm, dst_g, ones_v, cs, cnt_sh):
    c = lax.axis_index("c")
    s = lax.axis_index("s")
    wid = c * NS + s

    pltpu.sync_copy(z_hbm, cnt_sh.at[pl.ds(s * RPT, RPT)])
    pltpu.sync_copy(ones_hbm, ones_v)
    plsc.subcore_barrier()

    pltpu.sync_copy(dst_hbm.at[pl.ds(wid * NCHUNK, NCHUNK)], dst_g)
    descs = []
    for b in range(NCHUNK):
        if b >= 8:
            descs[b - 8].wait()
        descs.append(
            pltpu.async_copy(ones_v, cnt_sh.at[dst_g.at[b]], cs, add=True))
    for d in descs[-8:]:
        d.wait()
    plsc.subcore_barrier()

    pltpu.sync_copy(cnt_sh.at[pl.ds(s * RPT, RPT)], cnt_hbm.at[c, pl.ds(s * RPT, RPT)])


def _sc_mesh():
    return plsc.VectorSubcoreMesh(
        core_axis_name="c", subcore_axis_name="s", num_cores=NC, num_subcores=NS
    )


@functools.lru_cache(maxsize=None)
def _make_seg_sum():
    return pl.kernel(
        _seg_body,
        out_type=[jax.ShapeDtypeStruct((NC, NP, D), jnp.float32)],
        mesh=_sc_mesh(),
        scratch_types=[
            pltpu.VMEM((NIB * GRP * CHUNK,), jnp.int32),
            pltpu.VMEM((NIB * GRP, CHUNK), jnp.int32),
            pltpu.VMEM((CHUNK, D), jnp.float32),
            pltpu.VMEM((CHUNK, D), jnp.float32),
            pltpu.SemaphoreType.DMA,
            pltpu.SemaphoreType.DMA,
            pltpu.SemaphoreType.DMA,
            pltpu.SemaphoreType.DMA,
            pltpu.VMEM_SHARED((NP, D), jnp.float32),
        ],
    )


@functools.lru_cache(maxsize=None)
def _make_counts():
    return pl.kernel(
        _cnt_body,
        out_type=[jax.ShapeDtypeStruct((NC, NP, D), jnp.float32)],
        mesh=_sc_mesh(),
        scratch_types=[
            pltpu.VMEM((NCHUNK, CHUNK), jnp.int32),
            pltpu.VMEM((CHUNK, D), jnp.float32),
            pltpu.SemaphoreType.DMA,
            pltpu.VMEM_SHARED((NP, D), jnp.float32),
        ],
    )


def _tc_body(relu, parts_ref, cnts_ref, x_ref, wl_ref, bl_ref, wr_ref, out_ref):
    summed = parts_ref[0] + parts_ref[1]
    cnt = cnts_ref[0] + cnts_ref[1]
    mean = summed * (1.0 / jnp.maximum(cnt, 1.0))
    h = (jnp.dot(mean, wl_ref[...], preferred_element_type=jnp.float32)
         + bl_ref[...]
         + jnp.dot(x_ref[...], wr_ref[...], preferred_element_type=jnp.float32))
    out_ref[...] = jnp.maximum(h, 0.0) if relu else h


BN = 1024


def _sage_tc(parts, cnts, x, wl, bl, wr, relu):
    grid = NP // BN
    return pl.pallas_call(
        functools.partial(_tc_body, relu),
        grid=(grid,),
        in_specs=[
            pl.BlockSpec((NC, BN, D), lambda j: (0, j, 0)),
            pl.BlockSpec((NC, BN, 1), lambda j: (0, j, 0)),
            pl.BlockSpec((BN, D), lambda j: (j, 0)),
            pl.BlockSpec((D, D), lambda j: (0, 0)),
            pl.BlockSpec((1, D), lambda j: (0, 0)),
            pl.BlockSpec((D, D), lambda j: (0, 0)),
        ],
        out_specs=pl.BlockSpec((BN, D), lambda j: (j, 0)),
        out_shape=jax.ShapeDtypeStruct((N_NODES, D), jnp.float32),
    )(parts, cnts, x, wl, bl.reshape(1, D), wr)


def kernel(x, edge_index, Wl1, bl1, Wr1, Wl2, bl2, Wr2):
    src = edge_index[0]
    dst = edge_index[1]
    pad = EP - N_EDGES
    pad_ids = jnp.arange(pad, dtype=jnp.int32)
    src_p = jnp.concatenate([src, pad_ids % N_NODES])
    dst_p = jnp.concatenate([dst, N_NODES + pad_ids % (NP - N_NODES)])
    dst2d = dst_p.reshape(EP // CHUNK, CHUNK)

    zeros_rows = jnp.zeros((RPT, D), jnp.float32)
    ones_rows = jnp.ones((CHUNK, D), jnp.float32)

    cnts = _make_counts()(dst2d, zeros_rows, ones_rows)[0][:, :, 0:1]
    parts1 = _make_seg_sum()(x, src_p, dst2d, zeros_rows)[0]
    h = _sage_tc(parts1, cnts, x, Wl1, bl1, Wr1, relu=True)
    parts2 = _make_seg_sum()(h, src_p, dst2d, zeros_rows)[0]
    out = _sage_tc(parts2, cnts, h, Wl2, bl2, Wr2, relu=False)
    return out
